# baseline (speedup 1.0000x reference)
import jax
import jax.numpy as jnp
from jax.experimental import pallas as pl
from jax.experimental.pallas import tpu as pltpu

_BT = 256


def _body(len_ref, x_ref, marker_ref, out_ref):
    b = pl.program_id(0)
    j = pl.program_id(1)
    l = len_ref[b]
    rows = j * _BT + jax.lax.broadcasted_iota(jnp.int32, (_BT, 1), 0)
    out = jnp.where(rows < l, x_ref[0], 0.0)
    out = jnp.where(rows == l, marker_ref[0, :][None, :], out)
    out_ref[0] = out


def kernel(x, length, marker):
    b, t, f = x.shape
    nj = (t + 1 + _BT - 1) // _BT
    length = length.astype(jnp.int32)
    marker2d = marker.reshape(1, f)

    grid_spec = pltpu.PrefetchScalarGridSpec(
        num_scalar_prefetch=1,
        grid=(b, nj),
        in_specs=[
            pl.BlockSpec(
                (1, _BT, f),
                lambda bi, j, len_ref: (
                    bi,
                    jnp.where(j * _BT < len_ref[bi], j, 0),
                    0,
                ),
            ),
            pl.BlockSpec((1, f), lambda bi, j, len_ref: (0, 0)),
        ],
        out_specs=pl.BlockSpec((1, _BT, f), lambda bi, j, len_ref: (bi, j, 0)),
    )

    x_eos = pl.pallas_call(
        _body,
        grid_spec=grid_spec,
        out_shape=jax.ShapeDtypeStruct((b, t + 1, f), x.dtype),
    )(length, x, marker2d)

    length_eos = length.astype(jnp.float32) + 1.0
    return x_eos, length_eos


# speedup vs baseline: 1.0998x; 1.0998x over previous
import jax
import jax.numpy as jnp
from jax.experimental import pallas as pl
from jax.experimental.pallas import tpu as pltpu

_BT = 512


def _body(len_ref, x_ref, marker_ref, out_ref):
    b = pl.program_id(0)
    j = pl.program_id(1)
    l = len_ref[b]
    rows = j * _BT + jax.lax.broadcasted_iota(jnp.int32, (_BT, 1), 0)
    out = jnp.where(rows < l, x_ref[0], 0.0)
    out = jnp.where(rows == l, marker_ref[0, :][None, :], out)
    out_ref[0] = out


def kernel(x, length, marker):
    b, t, f = x.shape
    nj = (t + 1 + _BT - 1) // _BT
    length = length.astype(jnp.int32)
    marker2d = marker.reshape(1, f)

    grid_spec = pltpu.PrefetchScalarGridSpec(
        num_scalar_prefetch=1,
        grid=(b, nj),
        in_specs=[
            pl.BlockSpec(
                (1, _BT, f),
                lambda bi, j, len_ref: (
                    bi,
                    jnp.where(j * _BT < len_ref[bi], j, 0),
                    0,
                ),
            ),
            pl.BlockSpec((1, f), lambda bi, j, len_ref: (0, 0)),
        ],
        out_specs=pl.BlockSpec((1, _BT, f), lambda bi, j, len_ref: (bi, j, 0)),
    )

    x_eos = pl.pallas_call(
        _body,
        grid_spec=grid_spec,
        out_shape=jax.ShapeDtypeStruct((b, t + 1, f), x.dtype),
    )(length, x, marker2d)

    length_eos = length.astype(jnp.float32) + 1.0
    return x_eos, length_eos
